# Initial kernel scaffold; baseline (speedup 1.0000x reference)
import jax
import jax.numpy as jnp
from jax.experimental import pallas as pl


def kernel(x, edge_index, W1, b1, W2, b2, W3, b3):
    raise NotImplementedError("write your pallas kernel here")


# speedup vs baseline: 10.5158x; 10.5158x over previous
import functools

import jax
import jax.numpy as jnp
from jax import lax
from jax.experimental import pallas as pl
from jax.experimental.pallas import tpu as pltpu
from jax.experimental.pallas import tpu_sc as plsc

N = 10000
E = 320000
IN_DIM = 128
H1 = 128
H2 = 128
OUT = 64

NC = 2
NS = 16
NW = NC * NS
K = 128
CHUNKS = -(-E // (NW * K))
EPW = K * CHUNKS
EP = NW * EPW
NACC = 10240
RPT = NACC // NS
DUMMY = NACC - 8
DEG_D = 16
RB = 1024
GRID = NACC // RB

_mesh = lambda: plsc.VectorSubcoreMesh(
    core_axis_name="c", subcore_axis_name="s", num_cores=NC)


def _make_sc_agg(D):
    @functools.partial(
        pl.kernel,
        out_type=jax.ShapeDtypeStruct((NC, NACC, D), jnp.float32),
        mesh=_mesh(),
        scratch_types=[
            pltpu.VMEM((K,), jnp.int32),
            pltpu.VMEM((K,), jnp.int32),
            pltpu.VMEM((K, D), jnp.float32),
            pltpu.VMEM_SHARED((NACC, D), jnp.float32),
            pltpu.SemaphoreType.DMA,
        ],
        compiler_params=pltpu.CompilerParams(use_tc_tiling_on_sc=False),
    )
    def agg(h_hbm, src_hbm, dst_hbm, zeros_hbm, out_hbm,
            sidx, didx, rows, acc, sem):
        c = lax.axis_index("c")
        s = lax.axis_index("s")
        r0 = s * RPT
        pltpu.sync_copy(zeros_hbm, acc.at[pl.ds(r0, RPT)])
        plsc.subcore_barrier()
        base = (c * NS + s) * EPW

        def chunk(i, carry):
            off = base + i * K
            pltpu.sync_copy(src_hbm.at[pl.ds(off, K)], sidx)
            pltpu.sync_copy(dst_hbm.at[pl.ds(off, K)], didx)
            pltpu.async_copy(h_hbm.at[sidx], rows, sem).wait()
            pltpu.sync_copy(rows, acc.at[didx], add=True)
            return carry

        lax.fori_loop(0, CHUNKS, chunk, 0)
        plsc.subcore_barrier()
        pltpu.sync_copy(acc.at[pl.ds(r0, RPT)], out_hbm.at[c, pl.ds(r0, RPT)])

    return agg


_sc_agg128 = _make_sc_agg(128)
_sc_agg64 = _make_sc_agg(OUT)


@functools.partial(
    pl.kernel,
    out_type=jax.ShapeDtypeStruct((NC, NACC, DEG_D), jnp.float32),
    mesh=_mesh(),
    scratch_types=[
        pltpu.VMEM((K,), jnp.int32),
        pltpu.VMEM((K, DEG_D), jnp.float32),
        pltpu.VMEM_SHARED((NACC, DEG_D), jnp.float32),
    ],
    compiler_params=pltpu.CompilerParams(use_tc_tiling_on_sc=False),
)
def _sc_deg(dst_hbm, ones_hbm, zeros_hbm, out_hbm, didx, ones_v, acc):
    c = lax.axis_index("c")
    s = lax.axis_index("s")
    r0 = s * RPT
    pltpu.sync_copy(ones_hbm, ones_v)
    pltpu.sync_copy(zeros_hbm, acc.at[pl.ds(r0, RPT)])
    plsc.subcore_barrier()
    base = (c * NS + s) * EPW

    def chunk(i, carry):
        off = base + i * K
        pltpu.sync_copy(dst_hbm.at[pl.ds(off, K)], didx)
        pltpu.sync_copy(ones_v, acc.at[didx], add=True)
        return carry

    lax.fori_loop(0, CHUNKS, chunk, 0)
    plsc.subcore_barrier()
    pltpu.sync_copy(acc.at[pl.ds(r0, RPT)], out_hbm.at[c, pl.ds(r0, RPT)])


def _tc_first(x_p, p0, p1, W1):
    def body(x_ref, p0_ref, p1_ref, w_ref, g_ref, dv_ref):
        deg = p0_ref[...] + p1_ref[...] + 1.0
        dv = lax.rsqrt(deg)
        dv_ref[...] = dv
        g_ref[...] = jnp.dot(x_ref[...], w_ref[...],
                             preferred_element_type=jnp.float32) * dv[:, 0:1]

    return pl.pallas_call(
        body,
        grid=(GRID,),
        in_specs=[
            pl.BlockSpec((RB, IN_DIM), lambda i: (i, 0)),
            pl.BlockSpec((RB, DEG_D), lambda i: (i, 0)),
            pl.BlockSpec((RB, DEG_D), lambda i: (i, 0)),
            pl.BlockSpec((IN_DIM, H1), lambda i: (0, 0)),
        ],
        out_specs=[
            pl.BlockSpec((RB, H1), lambda i: (i, 0)),
            pl.BlockSpec((RB, DEG_D), lambda i: (i, 0)),
        ],
        out_shape=[
            jax.ShapeDtypeStruct((NACC, H1), jnp.float32),
            jax.ShapeDtypeStruct((NACC, DEG_D), jnp.float32),
        ],
    )(x_p, p0, p1, W1)


def _tc_mid(sa, sb, g, dv, b, W, d_in, d_out):
    def body(sa_ref, sb_ref, g_ref, dv_ref, b_ref, w_ref, o_ref):
        dvc = dv_ref[...][:, 0:1]
        z = jnp.maximum(
            dvc * (sa_ref[...] + sb_ref[...] + g_ref[...]) + b_ref[...], 0.0)
        o_ref[...] = jnp.dot(z, w_ref[...],
                             preferred_element_type=jnp.float32) * dvc

    return pl.pallas_call(
        body,
        grid=(GRID,),
        in_specs=[
            pl.BlockSpec((RB, d_in), lambda i: (i, 0)),
            pl.BlockSpec((RB, d_in), lambda i: (i, 0)),
            pl.BlockSpec((RB, d_in), lambda i: (i, 0)),
            pl.BlockSpec((RB, DEG_D), lambda i: (i, 0)),
            pl.BlockSpec((1, d_in), lambda i: (0, 0)),
            pl.BlockSpec((d_in, d_out), lambda i: (0, 0)),
        ],
        out_specs=pl.BlockSpec((RB, d_out), lambda i: (i, 0)),
        out_shape=jax.ShapeDtypeStruct((NACC, d_out), jnp.float32),
    )(sa, sb, g, dv, b, W)


def _tc_final(sa, sb, g, dv, b):
    def body(sa_ref, sb_ref, g_ref, dv_ref, b_ref, o_ref):
        dvc = dv_ref[...][:, 0:1]
        z = dvc * (sa_ref[...] + sb_ref[...] + g_ref[...]) + b_ref[...]
        m = jnp.max(z, axis=1, keepdims=True)
        e = jnp.exp(z - m)
        o_ref[...] = z - m - jnp.log(jnp.sum(e, axis=1, keepdims=True))

    return pl.pallas_call(
        body,
        grid=(GRID,),
        in_specs=[
            pl.BlockSpec((RB, OUT), lambda i: (i, 0)),
            pl.BlockSpec((RB, OUT), lambda i: (i, 0)),
            pl.BlockSpec((RB, OUT), lambda i: (i, 0)),
            pl.BlockSpec((RB, DEG_D), lambda i: (i, 0)),
            pl.BlockSpec((1, OUT), lambda i: (0, 0)),
        ],
        out_specs=pl.BlockSpec((RB, OUT), lambda i: (i, 0)),
        out_shape=jax.ShapeDtypeStruct((NACC, OUT), jnp.float32),
    )(sa, sb, g, dv, b)


def kernel(x, edge_index, W1, b1, W2, b2, W3, b3):
    src = edge_index[0]
    dst = edge_index[1]
    pad = EP - E
    src_p = jnp.concatenate([src, jnp.zeros((pad,), jnp.int32)])
    dst_p = jnp.concatenate([dst, jnp.full((pad,), DUMMY, jnp.int32)])
    ones16 = jnp.ones((K, DEG_D), jnp.float32)
    z16 = jnp.zeros((RPT, DEG_D), jnp.float32)
    z128 = jnp.zeros((RPT, 128), jnp.float32)
    z64 = jnp.zeros((RPT, OUT), jnp.float32)
    x_p = jnp.pad(x, ((0, NACC - N), (0, 0)))

    degP = _sc_deg(dst_p, ones16, z16)
    g1, dv = _tc_first(x_p, degP[0], degP[1], W1)
    S1 = _sc_agg128(g1, src_p, dst_p, z128)
    g2 = _tc_mid(S1[0], S1[1], g1, dv, b1.reshape(1, -1), W2, H1, H2)
    S2 = _sc_agg128(g2, src_p, dst_p, z128)
    g3 = _tc_mid(S2[0], S2[1], g2, dv, b2.reshape(1, -1), W3, H2, OUT)
    S3 = _sc_agg64(g3, src_p, dst_p, z64)
    out = _tc_final(S3[0], S3[1], g3, dv, b3.reshape(1, -1))
    return out[:N]
